# baseline (speedup 1.0000x reference)
import jax
import jax.numpy as jnp
import numpy as np
from jax import lax
from jax.experimental import pallas as pl
from jax.experimental.pallas import tpu as pltpu
from jax.experimental.pallas import tpu_sc as plsc

INPUT_DIM = 3
NUM_LEVELS = 16
FEATS = 2
LOG2_HASHMAP = 19
HASHMAP_SIZE = 2 ** LOG2_HASHMAP
MASK = HASHMAP_SIZE - 1
BASE_RES = 16
N_POINTS = 524288

_PRIMES_I32 = [int(x) for x in
               np.array([1958374283, 2654435761, 805459861],
                        dtype=np.uint64).astype(np.uint32).astype(np.int32)]

NW = 32
P = 256
NPW = N_POINTS // NW
CHUNKS = NPW // P
GROUPS = P // 16
DMA_ROWS = 2 * GROUPS


def _body(x_hbm, emb_hbm, out_hbm, xb, idxb, hb, rv, wb, ob,
          sem0, sem1):
    i32 = jnp.int32
    f32 = jnp.float32
    wid = lax.axis_index("c") * 16 + lax.axis_index("s")
    iota = lax.broadcasted_iota(i32, (16,), 0)
    zero16 = jnp.zeros((16,), i32)
    ones_f = jnp.full((16,), 1.0, f32)
    sfv0 = [c * 32 + iota for c in range(8)]
    sfv1 = [c * 32 + 16 + iota for c in range(8)]
    sems = (sem0, sem1)

    @pl.loop(jnp.int32(0), jnp.int32(CHUNKS))
    def _chunk(chunk):
        chunk = chunk.astype(jnp.int32)
        base = wid * NPW + chunk * P
        pltpu.sync_copy(x_hbm.at[pl.ds(base, P)], xb)

        def hash_and_fire(l, par):
            sem = sems[par]
            par = jnp.int32(par)
            res_f = jnp.left_shift(i32(BASE_RES), l).astype(f32)
            loff = jnp.left_shift(l, i32(LOG2_HASHMAP))

            @pl.loop(jnp.int32(0), jnp.int32(GROUPS))
            def _hash(g):
                g = g.astype(jnp.int32)
                pvec = g * 16 + iota
                c0 = zero16
                x0 = plsc.load_gather(xb, [pvec, c0])
                x1 = plsc.load_gather(xb, [pvec, c0 + 1])
                x2 = plsc.load_gather(xb, [pvec, c0 + 2])
                pos0 = x0 * res_f
                pos1 = x1 * res_f
                pos2 = x2 * res_f
                i0 = pos0.astype(i32)
                i1 = pos1.astype(i32)
                i2 = pos2.astype(i32)
                f0 = pos0 - i0.astype(f32)
                f1 = pos1 - i1.astype(f32)
                f2 = pos2 - i2.astype(f32)
                a0 = i0 * _PRIMES_I32[0]
                a1 = i1 * _PRIMES_I32[1]
                a2 = i2 * _PRIMES_I32[2]
                am0 = (a0 & MASK) + loff
                bm0 = ((a0 + _PRIMES_I32[0]) & MASK) + loff
                am1 = a1 & MASK
                bm1 = (a1 + _PRIMES_I32[1]) & MASK
                am2 = a2 & MASK
                bm2 = (a2 + _PRIMES_I32[2]) & MASK
                t00 = am0 ^ am1
                t10 = bm0 ^ am1
                t01 = am0 ^ bm1
                t11 = bm0 ^ bm1
                hs = [t00 ^ am2, t10 ^ am2, t01 ^ am2, t11 ^ am2,
                      t00 ^ bm2, t10 ^ bm2, t01 ^ bm2, t11 ^ bm2]
                for c in range(8):
                    h = hs[c]
                    hb[par, g, pl.ds(c * 16, 16)] = h
                    r = (jnp.left_shift(jnp.right_shift(h, i32(7)), i32(5))
                         | (jnp.right_shift(h, i32(3)) & 15))
                    row = 2 * g + (c // 4)
                    colb = (c % 4) * 32
                    idxb[par, row, pl.ds(colb, 16)] = r
                    idxb[par, row, pl.ds(colb + 16, 16)] = r + 16
                g0 = ones_f - f0
                g1 = ones_f - f1
                g2 = ones_f - f2
                u00 = g0 * g1
                u10 = f0 * g1
                u01 = g0 * f1
                u11 = f0 * f1
                off = g * 16
                wb[par, 0, pl.ds(off, 16)] = u00 * g2
                wb[par, 1, pl.ds(off, 16)] = u10 * g2
                wb[par, 2, pl.ds(off, 16)] = u01 * g2
                wb[par, 3, pl.ds(off, 16)] = u11 * g2
                wb[par, 4, pl.ds(off, 16)] = u00 * f2
                wb[par, 5, pl.ds(off, 16)] = u10 * f2
                wb[par, 6, pl.ds(off, 16)] = u01 * f2
                wb[par, 7, pl.ds(off, 16)] = u11 * f2

            @pl.loop(jnp.int32(0), jnp.int32(DMA_ROWS))
            def _gather(r):
                r = r.astype(jnp.int32)
                pltpu.async_copy(emb_hbm.at[idxb.at[par, r]],
                                 rv.at[par, pl.ds(r * 128, 128)], sem)

        def drain_and_accum(l, par):
            sem = sems[par]
            par = jnp.int32(par)

            pltpu.make_async_copy(emb_hbm.at[pl.ds(0, DMA_ROWS * 128)],
                                  rv.at[par], sem).wait()

            row_f0 = 2 * l
            row_f1 = row_f0 + 1

            @pl.loop(jnp.int32(0), jnp.int32(GROUPS))
            def _accum(g):
                g = g.astype(jnp.int32)
                rowbase = g * 256
                pvec = g * 16 + iota
                acc0 = jnp.zeros((16,), f32)
                acc1 = jnp.zeros((16,), f32)
                for c in range(8):
                    w = wb[par, c, pl.ds(g * 16, 16)]
                    sub = hb[par, g, pl.ds(c * 16, 16)] & 7
                    v0 = plsc.load_gather(rv.at[par],
                                          [rowbase + sfv0[c], sub])
                    v1 = plsc.load_gather(rv.at[par],
                                          [rowbase + sfv1[c], sub])
                    acc0 = acc0 + w * v0
                    acc1 = acc1 + w * v1
                plsc.store_scatter(ob, [zero16 + row_f0, pvec], acc0)
                plsc.store_scatter(ob, [zero16 + row_f1, pvec], acc1)

        @pl.loop(jnp.int32(0), jnp.int32(NUM_LEVELS // 2))
        def _lvl2(i):
            i = i.astype(jnp.int32)
            l0 = 2 * i
            hash_and_fire(l0, 0)

            @pl.when(i > 0)
            def _():
                drain_and_accum(l0 - 1, 1)

            hash_and_fire(l0 + 1, 1)
            drain_and_accum(l0, 0)

        drain_and_accum(i32(NUM_LEVELS - 1), 1)
        pltpu.sync_copy(ob, out_hbm.at[:, pl.ds(base, P)])


@jax.jit
def _encode(x, embp):
    mesh = plsc.VectorSubcoreMesh(core_axis_name="c", subcore_axis_name="s",
                                  num_cores=2, num_subcores=16)
    return pl.kernel(
        _body,
        out_type=jax.ShapeDtypeStruct((NUM_LEVELS * FEATS, N_POINTS),
                                      jnp.float32),
        mesh=mesh,
        scratch_types=[
            pltpu.VMEM((P, INPUT_DIM), jnp.float32),
            pltpu.VMEM((2, DMA_ROWS, 128), jnp.int32),
            pltpu.VMEM((2, GROUPS, 128), jnp.int32),
            pltpu.VMEM((2, DMA_ROWS * 128, 8), jnp.float32),
            pltpu.VMEM((2, 8, P), jnp.float32),
            pltpu.VMEM((NUM_LEVELS * FEATS, P), jnp.float32),
            pltpu.SemaphoreType.DMA,
            pltpu.SemaphoreType.DMA,
        ],
        compiler_params=pltpu.CompilerParams(needs_layout_passes=False,
                                             use_tc_tiling_on_sc=False),
    )(x, embp)


def kernel(x, embeddings):
    embp = (embeddings.reshape(NUM_LEVELS, HASHMAP_SIZE // 128, 128, FEATS)
            .transpose(0, 1, 3, 2)
            .reshape(NUM_LEVELS * HASHMAP_SIZE * FEATS // 8, 8))
    return _encode(x, embp).T


# speedup vs baseline: 1.6825x; 1.6825x over previous
import jax
import jax.numpy as jnp
import numpy as np
from jax import lax
from jax.experimental import pallas as pl
from jax.experimental.pallas import tpu as pltpu
from jax.experimental.pallas import tpu_sc as plsc

INPUT_DIM = 3
NUM_LEVELS = 16
FEATS = 2
LOG2_HASHMAP = 19
HASHMAP_SIZE = 2 ** LOG2_HASHMAP
MASK = HASHMAP_SIZE - 1
BASE_RES = 16
N_POINTS = 524288

_PRIMES_I32 = [int(x) for x in
               np.array([1958374283, 2654435761, 805459861],
                        dtype=np.uint64).astype(np.uint32).astype(np.int32)]

NW = 32
P = 256
NPW = N_POINTS // NW
CHUNKS = NPW // P
GROUPS = P // 16
DMA_ROWS = GROUPS


def _body(x_hbm, emb_hbm, out_hbm, xb, idxb, hb, rv, wb, ob,
          sem0, sem1):
    i32 = jnp.int32
    f32 = jnp.float32
    wid = lax.axis_index("c") * 16 + lax.axis_index("s")
    iota = lax.broadcasted_iota(i32, (16,), 0)
    zero16 = jnp.zeros((16,), i32)
    ones_f = jnp.full((16,), 1.0, f32)
    sfv = [c * 16 + iota for c in range(8)]
    sems = (sem0, sem1)

    @pl.loop(jnp.int32(0), jnp.int32(CHUNKS))
    def _chunk(chunk):
        chunk = chunk.astype(jnp.int32)
        base = wid * NPW + chunk * P
        pltpu.sync_copy(x_hbm.at[pl.ds(base, P)], xb)

        def hash_and_fire(l, par):
            sem = sems[par]
            par = jnp.int32(par)
            res_f = jnp.left_shift(i32(BASE_RES), l).astype(f32)
            loff = jnp.left_shift(l, i32(LOG2_HASHMAP))

            @pl.loop(jnp.int32(0), jnp.int32(GROUPS))
            def _hash(g):
                g = g.astype(jnp.int32)
                pvec = g * 16 + iota
                c0 = zero16
                x0 = plsc.load_gather(xb, [pvec, c0])
                x1 = plsc.load_gather(xb, [pvec, c0 + 1])
                x2 = plsc.load_gather(xb, [pvec, c0 + 2])
                pos0 = x0 * res_f
                pos1 = x1 * res_f
                pos2 = x2 * res_f
                i0 = pos0.astype(i32)
                i1 = pos1.astype(i32)
                i2 = pos2.astype(i32)
                f0 = pos0 - i0.astype(f32)
                f1 = pos1 - i1.astype(f32)
                f2 = pos2 - i2.astype(f32)
                a0 = i0 * _PRIMES_I32[0]
                a1 = i1 * _PRIMES_I32[1]
                a2 = i2 * _PRIMES_I32[2]
                am0 = (a0 & MASK) + loff
                bm0 = ((a0 + _PRIMES_I32[0]) & MASK) + loff
                am1 = a1 & MASK
                bm1 = (a1 + _PRIMES_I32[1]) & MASK
                am2 = a2 & MASK
                bm2 = (a2 + _PRIMES_I32[2]) & MASK
                t00 = am0 ^ am1
                t10 = bm0 ^ am1
                t01 = am0 ^ bm1
                t11 = bm0 ^ bm1
                hs = [t00 ^ am2, t10 ^ am2, t01 ^ am2, t11 ^ am2,
                      t00 ^ bm2, t10 ^ bm2, t01 ^ bm2, t11 ^ bm2]
                for c in range(8):
                    h = hs[c]
                    hb[par, g, pl.ds(c * 16, 16)] = h
                    idxb[par, g, pl.ds(c * 16, 16)] = (
                        jnp.right_shift(h, i32(2)))
                g0 = ones_f - f0
                g1 = ones_f - f1
                g2 = ones_f - f2
                u00 = g0 * g1
                u10 = f0 * g1
                u01 = g0 * f1
                u11 = f0 * f1
                off = g * 16
                wb[par, 0, pl.ds(off, 16)] = u00 * g2
                wb[par, 1, pl.ds(off, 16)] = u10 * g2
                wb[par, 2, pl.ds(off, 16)] = u01 * g2
                wb[par, 3, pl.ds(off, 16)] = u11 * g2
                wb[par, 4, pl.ds(off, 16)] = u00 * f2
                wb[par, 5, pl.ds(off, 16)] = u10 * f2
                wb[par, 6, pl.ds(off, 16)] = u01 * f2
                wb[par, 7, pl.ds(off, 16)] = u11 * f2

            @pl.loop(jnp.int32(0), jnp.int32(DMA_ROWS))
            def _gather(r):
                r = r.astype(jnp.int32)
                pltpu.async_copy(emb_hbm.at[idxb.at[par, r]],
                                 rv.at[par, pl.ds(r * 128, 128)], sem)

        def drain_and_accum(l, par):
            sem = sems[par]
            par = jnp.int32(par)

            pltpu.make_async_copy(emb_hbm.at[pl.ds(0, DMA_ROWS * 128)],
                                  rv.at[par], sem).wait()

            row_f0 = 2 * l
            row_f1 = row_f0 + 1

            @pl.loop(jnp.int32(0), jnp.int32(GROUPS))
            def _accum(g):
                g = g.astype(jnp.int32)
                rowbase = g * 128
                pvec = g * 16 + iota
                acc0 = jnp.zeros((16,), f32)
                acc1 = jnp.zeros((16,), f32)
                for c in range(8):
                    w = wb[par, c, pl.ds(g * 16, 16)]
                    sub2 = jnp.left_shift(hb[par, g, pl.ds(c * 16, 16)] & 3,
                                          i32(1))
                    rows = rowbase + sfv[c]
                    v0 = plsc.load_gather(rv.at[par], [rows, sub2])
                    v1 = plsc.load_gather(rv.at[par], [rows, sub2 + 1])
                    acc0 = acc0 + w * v0
                    acc1 = acc1 + w * v1
                plsc.store_scatter(ob, [zero16 + row_f0, pvec], acc0)
                plsc.store_scatter(ob, [zero16 + row_f1, pvec], acc1)

        @pl.loop(jnp.int32(0), jnp.int32(NUM_LEVELS // 2))
        def _lvl2(i):
            i = i.astype(jnp.int32)
            l0 = 2 * i
            hash_and_fire(l0, 0)

            @pl.when(i > 0)
            def _():
                drain_and_accum(l0 - 1, 1)

            hash_and_fire(l0 + 1, 1)
            drain_and_accum(l0, 0)

        drain_and_accum(i32(NUM_LEVELS - 1), 1)
        pltpu.sync_copy(ob, out_hbm.at[:, pl.ds(base, P)])


_RL_BLOCKS = NUM_LEVELS * HASHMAP_SIZE // 128
_RL_PER_W = _RL_BLOCKS // NW
_RL_B = 16


def _relayout_body(src_hbm, dst_hbm, tb, tob):
    i32 = jnp.int32
    wid = lax.axis_index("c") * 16 + lax.axis_index("s")
    iota = lax.broadcasted_iota(i32, (16,), 0)
    iota2 = iota * 2
    zero16 = jnp.zeros((16,), i32)

    @pl.loop(jnp.int32(0), jnp.int32(_RL_PER_W // _RL_B))
    def _batch(t):
        t = t.astype(jnp.int32)
        b0 = wid * _RL_PER_W + t * _RL_B
        pltpu.sync_copy(src_hbm.at[pl.ds(b0, _RL_B)], tb)
        for blk in range(_RL_B):
            bvec = zero16 + blk
            for c16 in range(8):
                v0 = tb[blk, pl.ds(c16 * 16, 16)]
                v1 = tb[blk, pl.ds(128 + c16 * 16, 16)]
                idx0 = c16 * 32 + iota2
                plsc.store_scatter(tob, [bvec, idx0], v0)
                plsc.store_scatter(tob, [bvec, idx0 + 1], v1)
        pltpu.sync_copy(tob, dst_hbm.at[pl.ds(b0, _RL_B)])


@jax.jit
def _encode(x, embp):
    mesh = plsc.VectorSubcoreMesh(core_axis_name="c", subcore_axis_name="s",
                                  num_cores=2, num_subcores=16)
    emb_std = pl.kernel(
        _relayout_body,
        out_type=jax.ShapeDtypeStruct((_RL_BLOCKS, 256), jnp.float32),
        mesh=mesh,
        scratch_types=[
            pltpu.VMEM((_RL_B, 256), jnp.float32),
            pltpu.VMEM((_RL_B, 256), jnp.float32),
        ],
        compiler_params=pltpu.CompilerParams(needs_layout_passes=False,
                                             use_tc_tiling_on_sc=False),
    )(embp.reshape(_RL_BLOCKS, 256))
    emb_std = emb_std.reshape(NUM_LEVELS * HASHMAP_SIZE * FEATS // 8, 8)
    return _encode_main(x, emb_std)


@jax.jit
def _encode_main(x, embp):
    mesh = plsc.VectorSubcoreMesh(core_axis_name="c", subcore_axis_name="s",
                                  num_cores=2, num_subcores=16)
    return pl.kernel(
        _body,
        out_type=jax.ShapeDtypeStruct((NUM_LEVELS * FEATS, N_POINTS),
                                      jnp.float32),
        mesh=mesh,
        scratch_types=[
            pltpu.VMEM((P, INPUT_DIM), jnp.float32),
            pltpu.VMEM((2, DMA_ROWS, 128), jnp.int32),
            pltpu.VMEM((2, GROUPS, 128), jnp.int32),
            pltpu.VMEM((2, DMA_ROWS * 128, 8), jnp.float32),
            pltpu.VMEM((2, 8, P), jnp.float32),
            pltpu.VMEM((NUM_LEVELS * FEATS, P), jnp.float32),
            pltpu.SemaphoreType.DMA,
            pltpu.SemaphoreType.DMA,
        ],
        compiler_params=pltpu.CompilerParams(needs_layout_passes=False,
                                             use_tc_tiling_on_sc=False),
    )(x, embp)


def kernel(x, embeddings):
    embp = (embeddings.reshape(NUM_LEVELS, HASHMAP_SIZE // 128, 128, FEATS)
            .transpose(0, 1, 3, 2)
            .reshape(NUM_LEVELS * HASHMAP_SIZE * FEATS // 8, 8))
    return _encode(x, embp).T


# speedup vs baseline: 1.6885x; 1.0036x over previous
import jax
import jax.numpy as jnp
import numpy as np
from jax import lax
from jax.experimental import pallas as pl
from jax.experimental.pallas import tpu as pltpu
from jax.experimental.pallas import tpu_sc as plsc

INPUT_DIM = 3
NUM_LEVELS = 16
FEATS = 2
LOG2_HASHMAP = 19
HASHMAP_SIZE = 2 ** LOG2_HASHMAP
MASK = HASHMAP_SIZE - 1
BASE_RES = 16
N_POINTS = 524288

_PRIMES_I32 = [int(x) for x in
               np.array([1958374283, 2654435761, 805459861],
                        dtype=np.uint64).astype(np.uint32).astype(np.int32)]

NW = 32
P = 512
NPW = N_POINTS // NW
CHUNKS = NPW // P
GROUPS = P // 16
DMA_ROWS = GROUPS


def _body(x_hbm, emb_hbm, out_hbm, xb, idxb, hb, rv, wb, ob,
          sem0, sem1):
    i32 = jnp.int32
    f32 = jnp.float32
    wid = lax.axis_index("c") * 16 + lax.axis_index("s")
    iota = lax.broadcasted_iota(i32, (16,), 0)
    zero16 = jnp.zeros((16,), i32)
    ones_f = jnp.full((16,), 1.0, f32)
    sfv = [c * 16 + iota for c in range(8)]
    sems = (sem0, sem1)

    @pl.loop(jnp.int32(0), jnp.int32(CHUNKS))
    def _chunk(chunk):
        chunk = chunk.astype(jnp.int32)
        base = wid * NPW + chunk * P
        pltpu.sync_copy(x_hbm.at[pl.ds(base, P)], xb)

        def hash_and_fire(l, par):
            sem = sems[par]
            par = jnp.int32(par)
            res_f = jnp.left_shift(i32(BASE_RES), l).astype(f32)
            loff = jnp.left_shift(l, i32(LOG2_HASHMAP))

            @pl.loop(jnp.int32(0), jnp.int32(GROUPS))
            def _hash(g):
                g = g.astype(jnp.int32)
                pvec = g * 16 + iota
                c0 = zero16
                x0 = plsc.load_gather(xb, [pvec, c0])
                x1 = plsc.load_gather(xb, [pvec, c0 + 1])
                x2 = plsc.load_gather(xb, [pvec, c0 + 2])
                pos0 = x0 * res_f
                pos1 = x1 * res_f
                pos2 = x2 * res_f
                i0 = pos0.astype(i32)
                i1 = pos1.astype(i32)
                i2 = pos2.astype(i32)
                f0 = pos0 - i0.astype(f32)
                f1 = pos1 - i1.astype(f32)
                f2 = pos2 - i2.astype(f32)
                a0 = i0 * _PRIMES_I32[0]
                a1 = i1 * _PRIMES_I32[1]
                a2 = i2 * _PRIMES_I32[2]
                am0 = (a0 & MASK) + loff
                bm0 = ((a0 + _PRIMES_I32[0]) & MASK) + loff
                am1 = a1 & MASK
                bm1 = (a1 + _PRIMES_I32[1]) & MASK
                am2 = a2 & MASK
                bm2 = (a2 + _PRIMES_I32[2]) & MASK
                t00 = am0 ^ am1
                t10 = bm0 ^ am1
                t01 = am0 ^ bm1
                t11 = bm0 ^ bm1
                hs = [t00 ^ am2, t10 ^ am2, t01 ^ am2, t11 ^ am2,
                      t00 ^ bm2, t10 ^ bm2, t01 ^ bm2, t11 ^ bm2]
                for c in range(8):
                    h = hs[c]
                    hb[par, g, pl.ds(c * 16, 16)] = h
                    idxb[par, g, pl.ds(c * 16, 16)] = (
                        jnp.right_shift(h, i32(2)))
                g0 = ones_f - f0
                g1 = ones_f - f1
                g2 = ones_f - f2
                u00 = g0 * g1
                u10 = f0 * g1
                u01 = g0 * f1
                u11 = f0 * f1
                off = g * 16
                wb[par, 0, pl.ds(off, 16)] = u00 * g2
                wb[par, 1, pl.ds(off, 16)] = u10 * g2
                wb[par, 2, pl.ds(off, 16)] = u01 * g2
                wb[par, 3, pl.ds(off, 16)] = u11 * g2
                wb[par, 4, pl.ds(off, 16)] = u00 * f2
                wb[par, 5, pl.ds(off, 16)] = u10 * f2
                wb[par, 6, pl.ds(off, 16)] = u01 * f2
                wb[par, 7, pl.ds(off, 16)] = u11 * f2

            @pl.loop(jnp.int32(0), jnp.int32(DMA_ROWS))
            def _gather(r):
                r = r.astype(jnp.int32)
                pltpu.async_copy(emb_hbm.at[idxb.at[par, r]],
                                 rv.at[par, pl.ds(r * 128, 128)], sem)

        def drain_and_accum(l, par):
            sem = sems[par]
            par = jnp.int32(par)

            pltpu.make_async_copy(emb_hbm.at[pl.ds(0, DMA_ROWS * 128)],
                                  rv.at[par], sem).wait()

            row_f0 = 2 * l
            row_f1 = row_f0 + 1

            @pl.loop(jnp.int32(0), jnp.int32(GROUPS))
            def _accum(g):
                g = g.astype(jnp.int32)
                rowbase = g * 128
                pvec = g * 16 + iota
                acc0 = jnp.zeros((16,), f32)
                acc1 = jnp.zeros((16,), f32)
                for c in range(8):
                    w = wb[par, c, pl.ds(g * 16, 16)]
                    sub2 = jnp.left_shift(hb[par, g, pl.ds(c * 16, 16)] & 3,
                                          i32(1))
                    rows = rowbase + sfv[c]
                    v0 = plsc.load_gather(rv.at[par], [rows, sub2])
                    v1 = plsc.load_gather(rv.at[par], [rows, sub2 + 1])
                    acc0 = acc0 + w * v0
                    acc1 = acc1 + w * v1
                plsc.store_scatter(ob, [zero16 + row_f0, pvec], acc0)
                plsc.store_scatter(ob, [zero16 + row_f1, pvec], acc1)

        @pl.loop(jnp.int32(0), jnp.int32(NUM_LEVELS // 2))
        def _lvl2(i):
            i = i.astype(jnp.int32)
            l0 = 2 * i
            hash_and_fire(l0, 0)

            @pl.when(i > 0)
            def _():
                drain_and_accum(l0 - 1, 1)

            hash_and_fire(l0 + 1, 1)
            drain_and_accum(l0, 0)

        drain_and_accum(i32(NUM_LEVELS - 1), 1)
        pltpu.sync_copy(ob, out_hbm.at[:, pl.ds(base, P)])


_RL_BLOCKS = NUM_LEVELS * HASHMAP_SIZE // 128
_RL_PER_W = _RL_BLOCKS // NW
_RL_B = 16


def _relayout_body(src_hbm, dst_hbm, tb, tob):
    i32 = jnp.int32
    wid = lax.axis_index("c") * 16 + lax.axis_index("s")
    iota = lax.broadcasted_iota(i32, (16,), 0)
    iota2 = iota * 2
    zero16 = jnp.zeros((16,), i32)

    @pl.loop(jnp.int32(0), jnp.int32(_RL_PER_W // _RL_B))
    def _batch(t):
        t = t.astype(jnp.int32)
        b0 = wid * _RL_PER_W + t * _RL_B
        pltpu.sync_copy(src_hbm.at[pl.ds(b0, _RL_B)], tb)
        for blk in range(_RL_B):
            bvec = zero16 + blk
            for c16 in range(8):
                v0 = tb[blk, pl.ds(c16 * 16, 16)]
                v1 = tb[blk, pl.ds(128 + c16 * 16, 16)]
                idx0 = c16 * 32 + iota2
                plsc.store_scatter(tob, [bvec, idx0], v0)
                plsc.store_scatter(tob, [bvec, idx0 + 1], v1)
        pltpu.sync_copy(tob, dst_hbm.at[pl.ds(b0, _RL_B)])


@jax.jit
def _encode(x, embp):
    mesh = plsc.VectorSubcoreMesh(core_axis_name="c", subcore_axis_name="s",
                                  num_cores=2, num_subcores=16)
    emb_std = pl.kernel(
        _relayout_body,
        out_type=jax.ShapeDtypeStruct((_RL_BLOCKS, 256), jnp.float32),
        mesh=mesh,
        scratch_types=[
            pltpu.VMEM((_RL_B, 256), jnp.float32),
            pltpu.VMEM((_RL_B, 256), jnp.float32),
        ],
        compiler_params=pltpu.CompilerParams(needs_layout_passes=False,
                                             use_tc_tiling_on_sc=False),
    )(embp.reshape(_RL_BLOCKS, 256))
    emb_std = emb_std.reshape(NUM_LEVELS * HASHMAP_SIZE * FEATS // 8, 8)
    return _encode_main(x, emb_std)


@jax.jit
def _encode_main(x, embp):
    mesh = plsc.VectorSubcoreMesh(core_axis_name="c", subcore_axis_name="s",
                                  num_cores=2, num_subcores=16)
    return pl.kernel(
        _body,
        out_type=jax.ShapeDtypeStruct((NUM_LEVELS * FEATS, N_POINTS),
                                      jnp.float32),
        mesh=mesh,
        scratch_types=[
            pltpu.VMEM((P, INPUT_DIM), jnp.float32),
            pltpu.VMEM((2, DMA_ROWS, 128), jnp.int32),
            pltpu.VMEM((2, GROUPS, 128), jnp.int32),
            pltpu.VMEM((2, DMA_ROWS * 128, 8), jnp.float32),
            pltpu.VMEM((2, 8, P), jnp.float32),
            pltpu.VMEM((NUM_LEVELS * FEATS, P), jnp.float32),
            pltpu.SemaphoreType.DMA,
            pltpu.SemaphoreType.DMA,
        ],
        compiler_params=pltpu.CompilerParams(needs_layout_passes=False,
                                             use_tc_tiling_on_sc=False),
    )(x, embp)


def kernel(x, embeddings):
    embp = (embeddings.reshape(NUM_LEVELS, HASHMAP_SIZE // 128, 128, FEATS)
            .transpose(0, 1, 3, 2)
            .reshape(NUM_LEVELS * HASHMAP_SIZE * FEATS // 8, 8))
    return _encode(x, embp).T


# speedup vs baseline: 2.1006x; 1.2440x over previous
import jax
import jax.numpy as jnp
import numpy as np
from jax import lax
from jax.experimental import pallas as pl
from jax.experimental.pallas import tpu as pltpu
from jax.experimental.pallas import tpu_sc as plsc

INPUT_DIM = 3
NUM_LEVELS = 16
FEATS = 2
LOG2_HASHMAP = 19
HASHMAP_SIZE = 2 ** LOG2_HASHMAP
MASK = HASHMAP_SIZE - 1
BASE_RES = 16
N_POINTS = 524288

_PRIMES_I32 = [int(x) for x in
               np.array([1958374283, 2654435761, 805459861],
                        dtype=np.uint64).astype(np.uint32).astype(np.int32)]

NW = 32
P = 512
NPW = N_POINTS // NW
CHUNKS = NPW // P
GROUPS = P // 16
DMA_ROWS = GROUPS


def _body(x_hbm, emb_hbm, out_hbm, xb, idxb, hb, rv, wb, ob,
          sem0, sem1):
    i32 = jnp.int32
    f32 = jnp.float32
    wid = lax.axis_index("c") * 16 + lax.axis_index("s")
    iota = lax.broadcasted_iota(i32, (16,), 0)
    zero16 = jnp.zeros((16,), i32)
    ones_f = jnp.full((16,), 1.0, f32)
    sfv = [c * 16 + iota for c in range(8)]
    sems = (sem0, sem1)

    @pl.loop(jnp.int32(0), jnp.int32(CHUNKS))
    def _chunk(chunk):
        chunk = chunk.astype(jnp.int32)
        base = wid * NPW + chunk * P
        pltpu.sync_copy(x_hbm.at[:, pl.ds(base, P)], xb)

        def hash_and_fire(l, par):
            sem = sems[par]
            par = jnp.int32(par)
            res_f = jnp.left_shift(i32(BASE_RES), l).astype(f32)
            loff = jnp.left_shift(l, i32(LOG2_HASHMAP))

            @pl.loop(jnp.int32(0), jnp.int32(GROUPS))
            def _hash(g):
                g = g.astype(jnp.int32)
                x0 = xb[0, pl.ds(g * 16, 16)]
                x1 = xb[1, pl.ds(g * 16, 16)]
                x2 = xb[2, pl.ds(g * 16, 16)]
                pos0 = x0 * res_f
                pos1 = x1 * res_f
                pos2 = x2 * res_f
                i0 = pos0.astype(i32)
                i1 = pos1.astype(i32)
                i2 = pos2.astype(i32)
                f0 = pos0 - i0.astype(f32)
                f1 = pos1 - i1.astype(f32)
                f2 = pos2 - i2.astype(f32)
                a0 = i0 * _PRIMES_I32[0]
                a1 = i1 * _PRIMES_I32[1]
                a2 = i2 * _PRIMES_I32[2]
                am0 = (a0 & MASK) + loff
                bm0 = ((a0 + _PRIMES_I32[0]) & MASK) + loff
                am1 = a1 & MASK
                bm1 = (a1 + _PRIMES_I32[1]) & MASK
                am2 = a2 & MASK
                bm2 = (a2 + _PRIMES_I32[2]) & MASK
                t00 = am0 ^ am1
                t10 = bm0 ^ am1
                t01 = am0 ^ bm1
                t11 = bm0 ^ bm1
                hs = [t00 ^ am2, t10 ^ am2, t01 ^ am2, t11 ^ am2,
                      t00 ^ bm2, t10 ^ bm2, t01 ^ bm2, t11 ^ bm2]
                for c in range(8):
                    h = hs[c]
                    hb[par, g, pl.ds(c * 16, 16)] = h
                    idxb[par, g, pl.ds(c * 16, 16)] = (
                        jnp.right_shift(h, i32(2)))
                g0 = ones_f - f0
                g1 = ones_f - f1
                g2 = ones_f - f2
                u00 = g0 * g1
                u10 = f0 * g1
                u01 = g0 * f1
                u11 = f0 * f1
                off = g * 16
                wb[par, 0, pl.ds(off, 16)] = u00 * g2
                wb[par, 1, pl.ds(off, 16)] = u10 * g2
                wb[par, 2, pl.ds(off, 16)] = u01 * g2
                wb[par, 3, pl.ds(off, 16)] = u11 * g2
                wb[par, 4, pl.ds(off, 16)] = u00 * f2
                wb[par, 5, pl.ds(off, 16)] = u10 * f2
                wb[par, 6, pl.ds(off, 16)] = u01 * f2
                wb[par, 7, pl.ds(off, 16)] = u11 * f2

            @pl.loop(jnp.int32(0), jnp.int32(DMA_ROWS))
            def _gather(r):
                r = r.astype(jnp.int32)
                pltpu.async_copy(emb_hbm.at[idxb.at[par, r]],
                                 rv.at[par, pl.ds(r * 128, 128)], sem)

        def drain_and_accum(l, par):
            sem = sems[par]
            par = jnp.int32(par)

            pltpu.make_async_copy(emb_hbm.at[pl.ds(0, DMA_ROWS * 128)],
                                  rv.at[par], sem).wait()

            row_f0 = 2 * l
            row_f1 = row_f0 + 1

            @pl.loop(jnp.int32(0), jnp.int32(GROUPS))
            def _accum(g):
                g = g.astype(jnp.int32)
                rowbase = g * 128
                pvec = g * 16 + iota
                acc0 = jnp.zeros((16,), f32)
                acc1 = jnp.zeros((16,), f32)
                for c in range(8):
                    w = wb[par, c, pl.ds(g * 16, 16)]
                    sub2 = jnp.left_shift(hb[par, g, pl.ds(c * 16, 16)] & 3,
                                          i32(1))
                    rows = rowbase + sfv[c]
                    v0 = plsc.load_gather(rv.at[par], [rows, sub2])
                    v1 = plsc.load_gather(rv.at[par], [rows, sub2 + 1])
                    acc0 = acc0 + w * v0
                    acc1 = acc1 + w * v1
                plsc.store_scatter(ob, [zero16 + row_f0, pvec], acc0)
                plsc.store_scatter(ob, [zero16 + row_f1, pvec], acc1)

        @pl.loop(jnp.int32(0), jnp.int32(NUM_LEVELS // 2))
        def _lvl2(i):
            i = i.astype(jnp.int32)
            l0 = 2 * i
            hash_and_fire(l0, 0)

            @pl.when(i > 0)
            def _():
                drain_and_accum(l0 - 1, 1)

            hash_and_fire(l0 + 1, 1)
            drain_and_accum(l0, 0)

        drain_and_accum(i32(NUM_LEVELS - 1), 1)
        pltpu.sync_copy(ob, out_hbm.at[:, pl.ds(base, P)])


_RL_BLOCKS = NUM_LEVELS * HASHMAP_SIZE // 128
_RL_PER_W = _RL_BLOCKS // NW
_RL_B = 16


def _relayout_body(src_hbm, dst_hbm, tb, tob):
    i32 = jnp.int32
    wid = lax.axis_index("c") * 16 + lax.axis_index("s")
    iota = lax.broadcasted_iota(i32, (16,), 0)
    iota2 = iota * 2
    zero16 = jnp.zeros((16,), i32)

    @pl.loop(jnp.int32(0), jnp.int32(_RL_PER_W // _RL_B))
    def _batch(t):
        t = t.astype(jnp.int32)
        b0 = wid * _RL_PER_W + t * _RL_B
        pltpu.sync_copy(src_hbm.at[pl.ds(b0, _RL_B)], tb)
        for blk in range(_RL_B):
            bvec = zero16 + blk
            for c16 in range(8):
                v0 = tb[blk, pl.ds(c16 * 16, 16)]
                v1 = tb[blk, pl.ds(128 + c16 * 16, 16)]
                idx0 = c16 * 32 + iota2
                plsc.store_scatter(tob, [bvec, idx0], v0)
                plsc.store_scatter(tob, [bvec, idx0 + 1], v1)
        pltpu.sync_copy(tob, dst_hbm.at[pl.ds(b0, _RL_B)])


@jax.jit
def _encode(x, embp):
    mesh = plsc.VectorSubcoreMesh(core_axis_name="c", subcore_axis_name="s",
                                  num_cores=2, num_subcores=16)
    emb_std = pl.kernel(
        _relayout_body,
        out_type=jax.ShapeDtypeStruct((_RL_BLOCKS, 256), jnp.float32),
        mesh=mesh,
        scratch_types=[
            pltpu.VMEM((_RL_B, 256), jnp.float32),
            pltpu.VMEM((_RL_B, 256), jnp.float32),
        ],
        compiler_params=pltpu.CompilerParams(needs_layout_passes=False,
                                             use_tc_tiling_on_sc=False),
    )(embp.reshape(_RL_BLOCKS, 256))
    emb_std = emb_std.reshape(NUM_LEVELS * HASHMAP_SIZE * FEATS // 8, 8)
    return _encode_main(x, emb_std)


@jax.jit
def _encode_main(x, embp):
    mesh = plsc.VectorSubcoreMesh(core_axis_name="c", subcore_axis_name="s",
                                  num_cores=2, num_subcores=16)
    return pl.kernel(
        _body,
        out_type=jax.ShapeDtypeStruct((NUM_LEVELS * FEATS, N_POINTS),
                                      jnp.float32),
        mesh=mesh,
        scratch_types=[
            pltpu.VMEM((INPUT_DIM, P), jnp.float32),
            pltpu.VMEM((2, DMA_ROWS, 128), jnp.int32),
            pltpu.VMEM((2, GROUPS, 128), jnp.int32),
            pltpu.VMEM((2, DMA_ROWS * 128, 8), jnp.float32),
            pltpu.VMEM((2, 8, P), jnp.float32),
            pltpu.VMEM((NUM_LEVELS * FEATS, P), jnp.float32),
            pltpu.SemaphoreType.DMA,
            pltpu.SemaphoreType.DMA,
        ],
        compiler_params=pltpu.CompilerParams(needs_layout_passes=False,
                                             use_tc_tiling_on_sc=False),
    )(x, embp)


def kernel(x, embeddings):
    x = x.T
    embp = (embeddings.reshape(NUM_LEVELS, HASHMAP_SIZE // 128, 128, FEATS)
            .transpose(0, 1, 3, 2)
            .reshape(NUM_LEVELS * HASHMAP_SIZE * FEATS // 8, 8))
    return _encode(x, embp).T
